# Initial kernel scaffold; baseline (speedup 1.0000x reference)
import jax
import jax.numpy as jnp
from jax.experimental import pallas as pl


def kernel(t, x, edge_index, W1, as1, ad1, b1, W2, as2, ad2, b2, W3, as3, ad3, b3):
    raise NotImplementedError("write your pallas kernel here")


# speedup vs baseline: 13.7258x; 13.7258x over previous
import functools

import jax
import jax.numpy as jnp
from jax import lax
from jax.experimental import pallas as pl
from jax.experimental.pallas import tpu as pltpu
from jax.experimental.pallas import tpu_sc as plsc

N = 50000
E = 800000
NEG = 0.2
D = 64
WA = 48
WB = 32
DA = 46
DB = D - DA
NC = 2
NS = 16
HALF = N // NC
HALF_PAD = 25088
CHUNK = 400
NGRP = CHUNK // 16
NCHUNK = E // CHUNK
CPT = NCHUNK // NS
R = 1000
NBLK = N // R

_SC_PARAMS = pltpu.CompilerParams(use_tc_tiling_on_sc=False,
                                  needs_layout_passes=False)


def _dense_body(nblk, x_ref, w_ref, s_ref, d_ref,
                hxa_ref, hxb_ref, as_ref, ad_ref, c_ref, sm):
    i = pl.program_id(0)

    @pl.when(i == 0)
    def _init():
        sm[0] = -jnp.inf
        sm[1] = -jnp.inf
        c_ref[...] = jnp.zeros((1, 1), jnp.float32)

    h = jnp.dot(x_ref[...], w_ref[...], preferred_element_type=jnp.float32)
    asv = jnp.sum(h * s_ref[...], axis=1, keepdims=True)
    adv = jnp.sum(h * d_ref[...], axis=1, keepdims=True)
    ones = jnp.ones((h.shape[0], 1), jnp.float32)
    hxa_ref[...] = jnp.concatenate([h[:, :DA], ones, asv], axis=1)
    hxb_ref[...] = jnp.concatenate(
        [h[:, DA:], ones, asv,
         jnp.zeros((h.shape[0], WB - DB - 2), jnp.float32)], axis=1)
    as_ref[...] = asv
    ad_ref[...] = adv
    sm[0] = jnp.maximum(sm[0], jnp.max(asv))
    sm[1] = jnp.maximum(sm[1], jnp.max(adv))

    @pl.when(i == nblk - 1)
    def _fin():
        c_ref[...] = jnp.maximum(sm[0] + sm[1], 0.0).reshape(1, 1)


def _dense(x, W, att_s, att_d):
    din = x.shape[1]
    body = functools.partial(_dense_body, NBLK)
    return pl.pallas_call(
        body,
        grid=(NBLK,),
        in_specs=[
            pl.BlockSpec((R, din), lambda i: (i, 0)),
            pl.BlockSpec((din, D), lambda i: (0, 0)),
            pl.BlockSpec((1, D), lambda i: (0, 0)),
            pl.BlockSpec((1, D), lambda i: (0, 0)),
        ],
        out_specs=[
            pl.BlockSpec((R, WA), lambda i: (i, 0)),
            pl.BlockSpec((R, WB), lambda i: (i, 0)),
            pl.BlockSpec((R, 1), lambda i: (i, 0)),
            pl.BlockSpec((R, 1), lambda i: (i, 0)),
            pl.BlockSpec((1, 1), lambda i: (0, 0)),
        ],
        out_shape=[
            jax.ShapeDtypeStruct((N, WA), jnp.float32),
            jax.ShapeDtypeStruct((N, WB), jnp.float32),
            jax.ShapeDtypeStruct((N, 1), jnp.float32),
            jax.ShapeDtypeStruct((N, 1), jnp.float32),
            jax.ShapeDtypeStruct((1, 1), jnp.float32),
        ],
        scratch_shapes=[pltpu.SMEM((2,), jnp.float32)],
    )(x, W, att_s.reshape(1, D), att_d.reshape(1, D))


def _finish_body(relu, numa_ref, numb_ref, hxa_ref, hxb_ref, as_ref, ad_ref,
                 c_ref, b_ref, o_ref):
    z = as_ref[...] + ad_ref[...]
    z = jnp.where(z > 0, z, NEG * z)
    exs = jnp.exp(z - c_ref[...])
    h = jnp.concatenate([hxa_ref[:, :DA], hxb_ref[:, :DB]], axis=1)
    num = jnp.concatenate([numa_ref[:, :DA], numb_ref[:, :DB]], axis=1)
    den = numa_ref[:, DA:DA + 1]
    out = (num + exs * h) / (den + exs + 1e-16)
    out = out + b_ref[...]
    if relu:
        out = jnp.maximum(out, 0.0)
    o_ref[...] = out


def _finish(numa, numb, hxa, hxb, asv, adv, c, b, relu):
    body = functools.partial(_finish_body, relu)
    return pl.pallas_call(
        body,
        grid=(NBLK,),
        in_specs=[
            pl.BlockSpec((R, WA), lambda i: (i, 0)),
            pl.BlockSpec((R, WB), lambda i: (i, 0)),
            pl.BlockSpec((R, WA), lambda i: (i, 0)),
            pl.BlockSpec((R, WB), lambda i: (i, 0)),
            pl.BlockSpec((R, 1), lambda i: (i, 0)),
            pl.BlockSpec((R, 1), lambda i: (i, 0)),
            pl.BlockSpec((1, 1), lambda i: (0, 0)),
            pl.BlockSpec((1, D), lambda i: (0, 0)),
        ],
        out_specs=pl.BlockSpec((R, D), lambda i: (i, 0)),
        out_shape=jax.ShapeDtypeStruct((N, D), jnp.float32),
    )(numa, numb, hxa, hxb, asv, adv, c, b.reshape(1, D))


def _sc_body(width, acol, src_hbm, dst_hbm, hx_hbm, ad_hbm, c_hbm,
             num_out, ad_t, c_v, srcb, dstb, exb, doffb, gbuf, num_sp, semg):
    cid = lax.axis_index("c")
    sid = lax.axis_index("s")
    lo = cid * HALF
    hi = lo + HALF
    nk = width // 16

    pltpu.sync_copy(ad_hbm.at[pl.ds(lo, HALF)], ad_t)
    pltpu.sync_copy(c_hbm, c_v)
    cvec = c_v[...]
    lane = lax.iota(jnp.int32, 16)

    zrow = jnp.zeros((16,), jnp.float32)
    for j in range(16):
        for k in range(nk):
            gbuf[j, pl.ds(k * 16, 16)] = zrow

    def _zero(i, _):
        pltpu.sync_copy(gbuf.at[pl.ds(0, 16)],
                        num_sp.at[pl.ds(sid * 1568 + i * 16, 16)])
        return 0

    lax.fori_loop(0, 98, _zero, 0)
    plsc.subcore_barrier()

    def _chunk(ci, _):
        base = (ci * NS + sid) * CHUNK
        pltpu.sync_copy(src_hbm.at[pl.ds(base, CHUNK)], srcb)
        pltpu.sync_copy(dst_hbm.at[pl.ds(base, CHUNK)], dstb)

        cps = []
        for g in range(NGRP):
            s16 = srcb[pl.ds(g * 16, 16)]
            cps.append(pltpu.async_copy(
                hx_hbm.at[s16], gbuf.at[pl.ds(g * 16, 16)], semg))
        for cp in cps:
            cp.wait()

        for g in range(NGRP):
            d16 = dstb[pl.ds(g * 16, 16)]
            inr = (d16 >= lo) & (d16 < hi)
            doff = jnp.where(inr, d16 - lo, 0)
            av = plsc.load_gather(gbuf, [g * 16 + lane,
                                         jnp.full((16,), acol, jnp.int32)])
            dv = plsc.load_gather(ad_t, [doff])
            e = av + dv
            e = jnp.where(e > 0, e, NEG * e)
            ex = jnp.where(inr, jnp.exp(e - cvec), 0.0)
            exb[pl.ds(g * 16, 16)] = ex
            doffb[pl.ds(g * 16, 16)] = doff

        def _scale(j, _):
            exj = plsc.load_gather(exb, [jnp.broadcast_to(j, (16,))])
            for k in range(nk):
                sl = pl.ds(k * 16, 16)
                gbuf[j, sl] = gbuf[j, sl] * exj
            return 0

        lax.fori_loop(0, CHUNK, _scale, 0)

        for g in range(NGRP):
            dof16 = doffb[pl.ds(g * 16, 16)]
            pltpu.sync_copy(gbuf.at[pl.ds(g * 16, 16)],
                            num_sp.at[dof16], add=True)
        return 0

    lax.fori_loop(0, CPT, _chunk, 0)
    plsc.subcore_barrier()

    def _copy_rows(soff, cnt):
        pltpu.sync_copy(num_sp.at[pl.ds(soff, cnt)], gbuf.at[pl.ds(0, cnt)])
        pltpu.sync_copy(gbuf.at[pl.ds(0, cnt)],
                        num_out.at[pl.ds(lo + soff, cnt)])

    tbase = sid * 1560
    _copy_rows(tbase, 400)
    _copy_rows(tbase + 400, 400)
    _copy_rows(tbase + 800, 400)
    _copy_rows(tbase + 1200, 360)

    @pl.when(sid == 0)
    def _tail():
        _copy_rows(16 * 1560, 40)


def _sc_edge_pass(src, dst, hx, adv, c16, width, acol):
    mesh = plsc.VectorSubcoreMesh(core_axis_name="c", subcore_axis_name="s")
    body = functools.partial(_sc_body, width, acol)

    @functools.partial(
        pl.kernel,
        mesh=mesh,
        compiler_params=_SC_PARAMS,
        out_type=jax.ShapeDtypeStruct((N, width), jnp.float32),
        scratch_types=[
            pltpu.VMEM((HALF,), jnp.float32),
            pltpu.VMEM((16,), jnp.float32),
            pltpu.VMEM((CHUNK,), jnp.int32),
            pltpu.VMEM((CHUNK,), jnp.int32),
            pltpu.VMEM((CHUNK,), jnp.float32),
            pltpu.VMEM((CHUNK,), jnp.int32),
            pltpu.VMEM((CHUNK, width), jnp.float32),
            pltpu.VMEM_SHARED((HALF_PAD, width), jnp.float32),
            pltpu.SemaphoreType.DMA,
        ],
    )
    def k(src_hbm, dst_hbm, hx_hbm, ad_hbm, c_hbm, num_out,
          ad_t, c_v, srcb, dstb, exb, doffb, gbuf, nsp, semg):
        body(src_hbm, dst_hbm, hx_hbm, ad_hbm, c_hbm, num_out,
             ad_t, c_v, srcb, dstb, exb, doffb, gbuf, nsp, semg)

    return k(src, dst, hx, adv, c16)


def _layer(x, edge_src, edge_dst, W, att_s, att_d, b, relu):
    hxa, hxb, asv, adv, c = _dense(x, W, att_s, att_d)
    c16 = jnp.broadcast_to(c.reshape(()), (16,))
    adv1 = adv.reshape(N)
    numa = _sc_edge_pass(edge_src, edge_dst, hxa, adv1, c16, WA, DA + 1)
    numb = _sc_edge_pass(edge_src, edge_dst, hxb, adv1, c16, WB, DB + 1)
    return _finish(numa, numb, hxa, hxb, asv, adv, c, b, relu)


def kernel(t, x, edge_index, W1, as1, ad1, b1, W2, as2, ad2, b2,
           W3, as3, ad3, b3):
    src = edge_index[0]
    dst = edge_index[1]
    h = _layer(x, src, dst, W1, as1, ad1, b1, relu=True)
    h = _layer(h, src, dst, W2, as2, ad2, b2, relu=True)
    return _layer(h, src, dst, W3, as3, ad3, b3, relu=False)


# speedup vs baseline: 17.4713x; 1.2729x over previous
import functools

import jax
import jax.numpy as jnp
from jax import lax
from jax.experimental import pallas as pl
from jax.experimental.pallas import tpu as pltpu
from jax.experimental.pallas import tpu_sc as plsc

N = 50000
E = 800000
NEG = 0.2
D = 64
WA = 48
WB = 32
DA = 46
DB = D - DA
NC = 2
NS = 16
HALF = N // NC
HALF_PAD = 25088
CHUNK = 400
NGRP = CHUNK // 16
NCHUNK = E // CHUNK
CPT = NCHUNK // NS
R = 1000
NBLK = N // R

_SC_PARAMS = pltpu.CompilerParams(use_tc_tiling_on_sc=False,
                                  needs_layout_passes=False)


def _dense_body(nblk, x_ref, w_ref, s_ref, d_ref,
                hxa_ref, hxb_ref, as_ref, ad_ref, c_ref, sm):
    i = pl.program_id(0)

    @pl.when(i == 0)
    def _init():
        sm[0] = -jnp.inf
        sm[1] = -jnp.inf
        c_ref[...] = jnp.zeros((1, 1), jnp.float32)

    h = jnp.dot(x_ref[...], w_ref[...], preferred_element_type=jnp.float32)
    asv = jnp.sum(h * s_ref[...], axis=1, keepdims=True)
    adv = jnp.sum(h * d_ref[...], axis=1, keepdims=True)
    ones = jnp.ones((h.shape[0], 1), jnp.float32)
    hxa_ref[...] = jnp.concatenate([h[:, :DA], ones, asv], axis=1)
    hxb_ref[...] = jnp.concatenate(
        [h[:, DA:], ones, asv,
         jnp.zeros((h.shape[0], WB - DB - 2), jnp.float32)], axis=1)
    as_ref[...] = asv
    ad_ref[...] = adv
    sm[0] = jnp.maximum(sm[0], jnp.max(asv))
    sm[1] = jnp.maximum(sm[1], jnp.max(adv))

    @pl.when(i == nblk - 1)
    def _fin():
        c_ref[...] = jnp.maximum(sm[0] + sm[1], 0.0).reshape(1, 1)


def _dense(x, W, att_s, att_d):
    din = x.shape[1]
    body = functools.partial(_dense_body, NBLK)
    return pl.pallas_call(
        body,
        grid=(NBLK,),
        in_specs=[
            pl.BlockSpec((R, din), lambda i: (i, 0)),
            pl.BlockSpec((din, D), lambda i: (0, 0)),
            pl.BlockSpec((1, D), lambda i: (0, 0)),
            pl.BlockSpec((1, D), lambda i: (0, 0)),
        ],
        out_specs=[
            pl.BlockSpec((R, WA), lambda i: (i, 0)),
            pl.BlockSpec((R, WB), lambda i: (i, 0)),
            pl.BlockSpec((R, 1), lambda i: (i, 0)),
            pl.BlockSpec((R, 1), lambda i: (i, 0)),
            pl.BlockSpec((1, 1), lambda i: (0, 0)),
        ],
        out_shape=[
            jax.ShapeDtypeStruct((N, WA), jnp.float32),
            jax.ShapeDtypeStruct((N, WB), jnp.float32),
            jax.ShapeDtypeStruct((N, 1), jnp.float32),
            jax.ShapeDtypeStruct((N, 1), jnp.float32),
            jax.ShapeDtypeStruct((1, 1), jnp.float32),
        ],
        scratch_shapes=[pltpu.SMEM((2,), jnp.float32)],
    )(x, W, att_s.reshape(1, D), att_d.reshape(1, D))


def _finish_body(relu, numa_ref, numb_ref, hxa_ref, hxb_ref, as_ref, ad_ref,
                 c_ref, b_ref, o_ref):
    z = as_ref[...] + ad_ref[...]
    z = jnp.where(z > 0, z, NEG * z)
    exs = jnp.exp(z - c_ref[...])
    h = jnp.concatenate([hxa_ref[:, :DA], hxb_ref[:, :DB]], axis=1)
    num = jnp.concatenate([numa_ref[:, :DA], numb_ref[:, :DB]], axis=1)
    den = numa_ref[:, DA:DA + 1]
    out = (num + exs * h) / (den + exs + 1e-16)
    out = out + b_ref[...]
    if relu:
        out = jnp.maximum(out, 0.0)
    o_ref[...] = out


def _finish(numa, numb, hxa, hxb, asv, adv, c, b, relu):
    body = functools.partial(_finish_body, relu)
    return pl.pallas_call(
        body,
        grid=(NBLK,),
        in_specs=[
            pl.BlockSpec((R, WA), lambda i: (i, 0)),
            pl.BlockSpec((R, WB), lambda i: (i, 0)),
            pl.BlockSpec((R, WA), lambda i: (i, 0)),
            pl.BlockSpec((R, WB), lambda i: (i, 0)),
            pl.BlockSpec((R, 1), lambda i: (i, 0)),
            pl.BlockSpec((R, 1), lambda i: (i, 0)),
            pl.BlockSpec((1, 1), lambda i: (0, 0)),
            pl.BlockSpec((1, D), lambda i: (0, 0)),
        ],
        out_specs=pl.BlockSpec((R, D), lambda i: (i, 0)),
        out_shape=jax.ShapeDtypeStruct((N, D), jnp.float32),
    )(numa, numb, hxa, hxb, asv, adv, c, b.reshape(1, D))


def _sc_body(width, acol, src_hbm, dst_hbm, hx_hbm, ad_hbm, c_hbm,
             num_out, ad_t, c_v, srcb, dstb, exb, doffb, gbuf, num_sp,
             semg, sems):
    cid = lax.axis_index("c")
    sid = lax.axis_index("s")
    lo = cid * HALF
    hi = lo + HALF
    nk = width // 16

    pltpu.sync_copy(ad_hbm.at[pl.ds(lo, HALF)], ad_t)
    pltpu.sync_copy(c_hbm, c_v)
    cvec = c_v[...]
    lane = lax.iota(jnp.int32, 16)

    zrow = jnp.zeros((16,), jnp.float32)
    for j in range(16):
        for k in range(nk):
            gbuf[j, pl.ds(k * 16, 16)] = zrow

    def _zero(i, _):
        pltpu.sync_copy(gbuf.at[pl.ds(0, 16)],
                        num_sp.at[pl.ds(sid * 1568 + i * 16, 16)])
        return 0

    lax.fori_loop(0, 98, _zero, 0)
    plsc.subcore_barrier()

    def _chunk(ci, _):
        base = (ci * NS + sid) * CHUNK
        pltpu.sync_copy(src_hbm.at[pl.ds(base, CHUNK)], srcb)
        pltpu.sync_copy(dst_hbm.at[pl.ds(base, CHUNK)], dstb)

        cps = []
        for g in range(NGRP):
            s16 = srcb[pl.ds(g * 16, 16)]
            cps.append(pltpu.async_copy(
                hx_hbm.at[s16], gbuf.at[pl.ds(g * 16, 16)], semg))
        for cp in cps:
            cp.wait()

        for g in range(NGRP):
            d16 = dstb[pl.ds(g * 16, 16)]
            inr = (d16 >= lo) & (d16 < hi)
            doff = jnp.where(inr, d16 - lo, 0)
            av = plsc.load_gather(gbuf, [g * 16 + lane,
                                         jnp.full((16,), acol, jnp.int32)])
            dv = plsc.load_gather(ad_t, [doff])
            e = av + dv
            e = jnp.where(e > 0, e, NEG * e)
            ex = jnp.where(inr, jnp.exp(e - cvec), 0.0)
            exb[pl.ds(g * 16, 16)] = ex
            doffb[pl.ds(g * 16, 16)] = doff

        scps = []
        for g in range(NGRP):
            def _scale(j, _, g=g):
                exj = plsc.load_gather(exb, [jnp.broadcast_to(g * 16 + j,
                                                              (16,))])
                for k in range(nk):
                    sl = pl.ds(k * 16, 16)
                    gbuf[g * 16 + j, sl] = gbuf[g * 16 + j, sl] * exj
                return 0

            lax.fori_loop(0, 16, _scale, 0)
            dof16 = doffb[pl.ds(g * 16, 16)]
            scps.append(pltpu.async_copy(
                gbuf.at[pl.ds(g * 16, 16)], num_sp.at[dof16], sems,
                add=True))
        for cp in scps:
            cp.wait()
        return 0

    lax.fori_loop(0, CPT, _chunk, 0)
    plsc.subcore_barrier()

    def _copy_rows(soff, cnt):
        pltpu.sync_copy(num_sp.at[pl.ds(soff, cnt)], gbuf.at[pl.ds(0, cnt)])
        pltpu.sync_copy(gbuf.at[pl.ds(0, cnt)],
                        num_out.at[pl.ds(lo + soff, cnt)])

    tbase = sid * 1560
    _copy_rows(tbase, 400)
    _copy_rows(tbase + 400, 400)
    _copy_rows(tbase + 800, 400)
    _copy_rows(tbase + 1200, 360)

    @pl.when(sid == 0)
    def _tail():
        _copy_rows(16 * 1560, 40)


def _sc_edge_pass(src, dst, hx, adv, c16, width, acol):
    mesh = plsc.VectorSubcoreMesh(core_axis_name="c", subcore_axis_name="s")
    body = functools.partial(_sc_body, width, acol)

    @functools.partial(
        pl.kernel,
        mesh=mesh,
        compiler_params=_SC_PARAMS,
        out_type=jax.ShapeDtypeStruct((N, width), jnp.float32),
        scratch_types=[
            pltpu.VMEM((HALF,), jnp.float32),
            pltpu.VMEM((16,), jnp.float32),
            pltpu.VMEM((CHUNK,), jnp.int32),
            pltpu.VMEM((CHUNK,), jnp.int32),
            pltpu.VMEM((CHUNK,), jnp.float32),
            pltpu.VMEM((CHUNK,), jnp.int32),
            pltpu.VMEM((CHUNK, width), jnp.float32),
            pltpu.VMEM_SHARED((HALF_PAD, width), jnp.float32),
            pltpu.SemaphoreType.DMA,
            pltpu.SemaphoreType.DMA,
        ],
    )
    def k(src_hbm, dst_hbm, hx_hbm, ad_hbm, c_hbm, num_out,
          ad_t, c_v, srcb, dstb, exb, doffb, gbuf, nsp, semg, sems):
        body(src_hbm, dst_hbm, hx_hbm, ad_hbm, c_hbm, num_out,
             ad_t, c_v, srcb, dstb, exb, doffb, gbuf, nsp, semg, sems)

    return k(src, dst, hx, adv, c16)


def _layer(x, edge_src, edge_dst, W, att_s, att_d, b, relu):
    hxa, hxb, asv, adv, c = _dense(x, W, att_s, att_d)
    c16 = jnp.broadcast_to(c.reshape(()), (16,))
    adv1 = adv.reshape(N)
    numa = _sc_edge_pass(edge_src, edge_dst, hxa, adv1, c16, WA, DA + 1)
    numb = _sc_edge_pass(edge_src, edge_dst, hxb, adv1, c16, WB, DB + 1)
    return _finish(numa, numb, hxa, hxb, asv, adv, c, b, relu)


def kernel(t, x, edge_index, W1, as1, ad1, b1, W2, as2, ad2, b2,
           W3, as3, ad3, b3):
    src = edge_index[0]
    dst = edge_index[1]
    h = _layer(x, src, dst, W1, as1, ad1, b1, relu=True)
    h = _layer(h, src, dst, W2, as2, ad2, b2, relu=True)
    return _layer(h, src, dst, W3, as3, ad3, b3, relu=False)


# speedup vs baseline: 22.8601x; 1.3084x over previous
import functools

import jax
import jax.numpy as jnp
from jax import lax
from jax.experimental import pallas as pl
from jax.experimental.pallas import tpu as pltpu
from jax.experimental.pallas import tpu_sc as plsc

N = 50000
E = 800000
NEG = 0.2
D = 64
WA = 48
WB = 32
DA = 46
DB = D - DA
NC = 2
NS = 16
HALF = N // NC
HALF_PAD = 25088
CHUNK = 400
CBUF = 816
NGRP = CHUNK // 16
NCHUNK = E // CHUNK
CPT = NCHUNK // NS
R = 1000
NBLK = N // R

_SC_PARAMS = pltpu.CompilerParams(use_tc_tiling_on_sc=False,
                                  needs_layout_passes=False)


def _dense_body(nblk, x_ref, w_ref, s_ref, d_ref,
                hxa_ref, hxb_ref, as_ref, ad_ref, c_ref, sm):
    i = pl.program_id(0)

    @pl.when(i == 0)
    def _init():
        sm[0] = -jnp.inf
        sm[1] = -jnp.inf
        c_ref[...] = jnp.zeros((1, 1), jnp.float32)

    h = jnp.dot(x_ref[...], w_ref[...], preferred_element_type=jnp.float32)
    asv = jnp.sum(h * s_ref[...], axis=1, keepdims=True)
    adv = jnp.sum(h * d_ref[...], axis=1, keepdims=True)
    ones = jnp.ones((h.shape[0], 1), jnp.float32)
    hxa_ref[...] = jnp.concatenate([h[:, :DA], ones, asv], axis=1)
    hxb_ref[...] = jnp.concatenate(
        [h[:, DA:], ones, asv,
         jnp.zeros((h.shape[0], WB - DB - 2), jnp.float32)], axis=1)
    as_ref[...] = asv
    ad_ref[...] = adv
    sm[0] = jnp.maximum(sm[0], jnp.max(asv))
    sm[1] = jnp.maximum(sm[1], jnp.max(adv))

    @pl.when(i == nblk - 1)
    def _fin():
        c_ref[...] = jnp.maximum(sm[0] + sm[1], 0.0).reshape(1, 1)


def _dense(x, W, att_s, att_d):
    din = x.shape[1]
    body = functools.partial(_dense_body, NBLK)
    return pl.pallas_call(
        body,
        grid=(NBLK,),
        in_specs=[
            pl.BlockSpec((R, din), lambda i: (i, 0)),
            pl.BlockSpec((din, D), lambda i: (0, 0)),
            pl.BlockSpec((1, D), lambda i: (0, 0)),
            pl.BlockSpec((1, D), lambda i: (0, 0)),
        ],
        out_specs=[
            pl.BlockSpec((R, WA), lambda i: (i, 0)),
            pl.BlockSpec((R, WB), lambda i: (i, 0)),
            pl.BlockSpec((R, 1), lambda i: (i, 0)),
            pl.BlockSpec((R, 1), lambda i: (i, 0)),
            pl.BlockSpec((1, 1), lambda i: (0, 0)),
        ],
        out_shape=[
            jax.ShapeDtypeStruct((N, WA), jnp.float32),
            jax.ShapeDtypeStruct((N, WB), jnp.float32),
            jax.ShapeDtypeStruct((N, 1), jnp.float32),
            jax.ShapeDtypeStruct((N, 1), jnp.float32),
            jax.ShapeDtypeStruct((1, 1), jnp.float32),
        ],
        scratch_shapes=[pltpu.SMEM((2,), jnp.float32)],
    )(x, W, att_s.reshape(1, D), att_d.reshape(1, D))


def _finish_body(relu, numa_ref, numb_ref, hxa_ref, hxb_ref, as_ref, ad_ref,
                 c_ref, b_ref, o_ref):
    z = as_ref[...] + ad_ref[...]
    z = jnp.where(z > 0, z, NEG * z)
    exs = jnp.exp(z - c_ref[...])
    h = jnp.concatenate([hxa_ref[:, :DA], hxb_ref[:, :DB]], axis=1)
    num = jnp.concatenate([numa_ref[:, :DA], numb_ref[:, :DB]], axis=1)
    den = numa_ref[:, DA:DA + 1]
    out = (num + exs * h) / (den + exs + 1e-16)
    out = out + b_ref[...]
    if relu:
        out = jnp.maximum(out, 0.0)
    o_ref[...] = out


def _finish(numa, numb, hxa, hxb, asv, adv, c, b, relu):
    body = functools.partial(_finish_body, relu)
    return pl.pallas_call(
        body,
        grid=(NBLK,),
        in_specs=[
            pl.BlockSpec((R, WA), lambda i: (i, 0)),
            pl.BlockSpec((R, WB), lambda i: (i, 0)),
            pl.BlockSpec((R, WA), lambda i: (i, 0)),
            pl.BlockSpec((R, WB), lambda i: (i, 0)),
            pl.BlockSpec((R, 1), lambda i: (i, 0)),
            pl.BlockSpec((R, 1), lambda i: (i, 0)),
            pl.BlockSpec((1, 1), lambda i: (0, 0)),
            pl.BlockSpec((1, D), lambda i: (0, 0)),
        ],
        out_specs=pl.BlockSpec((R, D), lambda i: (i, 0)),
        out_shape=jax.ShapeDtypeStruct((N, D), jnp.float32),
    )(numa, numb, hxa, hxb, asv, adv, c, b.reshape(1, D))


def _sc_body(width, acol, src_hbm, dst_hbm, hx_hbm, ad_hbm, c_hbm,
             num_out, ad_t, c_v, srcb, dstb, exb, csrc, cdoff, gbuf, num_sp,
             semg, sems):
    cid = lax.axis_index("c")
    sid = lax.axis_index("s")
    lo = cid * HALF
    hi = lo + HALF
    nk = width // 16

    pltpu.sync_copy(ad_hbm.at[pl.ds(lo, HALF)], ad_t)
    pltpu.sync_copy(c_hbm, c_v)
    cvec = c_v[...]
    lane = lax.iota(jnp.int32, 16)

    zrow = jnp.zeros((16,), jnp.float32)
    for j in range(16):
        for k in range(nk):
            gbuf[j, pl.ds(k * 16, 16)] = zrow

    def _zero(i, _):
        pltpu.sync_copy(gbuf.at[pl.ds(0, 16)],
                        num_sp.at[pl.ds(sid * 1568 + i * 16, 16)])
        return 0

    lax.fori_loop(0, 98, _zero, 0)
    plsc.subcore_barrier()

    acol16 = jnp.full((16,), acol, jnp.int32)

    zi = jnp.zeros((16,), jnp.int32)
    for g in range(CBUF // 16):
        csrc[pl.ds(g * 16, 16)] = zi
        cdoff[pl.ds(g * 16, 16)] = zi

    def _flush(masked, off):
        cps = []
        for g in range(NGRP):
            s16 = csrc[pl.ds(g * 16, 16)]
            cps.append(pltpu.async_copy(
                hx_hbm.at[s16], gbuf.at[pl.ds(g * 16, 16)], semg))
        for cp in cps:
            cp.wait()
        for g in range(NGRP):
            dof = cdoff[pl.ds(g * 16, 16)]
            av = plsc.load_gather(gbuf, [g * 16 + lane, acol16])
            dv = plsc.load_gather(ad_t, [dof])
            e = av + dv
            e = jnp.where(e > 0, e, NEG * e)
            ex = jnp.exp(e - cvec)
            if masked:
                ex = jnp.where(g * 16 + lane < off, ex, 0.0)
            exb[pl.ds(g * 16, 16)] = ex
        scps = []
        for g in range(NGRP):
            def _scale(j, _, g=g):
                exj = plsc.load_gather(exb, [jnp.broadcast_to(g * 16 + j,
                                                              (16,))])
                for k in range(nk):
                    sl = pl.ds(k * 16, 16)
                    gbuf[g * 16 + j, sl] = gbuf[g * 16 + j, sl] * exj
                return 0

            lax.fori_loop(0, 16, _scale, 0)
            dof16 = cdoff[pl.ds(g * 16, 16)]
            scps.append(pltpu.async_copy(
                gbuf.at[pl.ds(g * 16, 16)], num_sp.at[dof16], sems,
                add=True))
        for cp in scps:
            cp.wait()

    def _chunk(ci, off):
        base = (ci * NS + sid) * CHUNK
        pltpu.sync_copy(src_hbm.at[pl.ds(base, CHUNK)], srcb)
        pltpu.sync_copy(dst_hbm.at[pl.ds(base, CHUNK)], dstb)

        for g in range(NGRP):
            s16 = srcb[pl.ds(g * 16, 16)]
            d16 = dstb[pl.ds(g * 16, 16)]
            inr = (d16 >= lo) & (d16 < hi)
            doff = jnp.where(inr, d16 - lo, 0)
            plsc.store_compressed(csrc.at[pl.ds(off, 16)], s16, mask=inr)
            plsc.store_compressed(cdoff.at[pl.ds(off, 16)], doff, mask=inr)
            off = off + jnp.sum(inr.astype(jnp.int32))

        @pl.when(off >= CHUNK)
        def _do_flush():
            _flush(False, 0)
            for g in range(NGRP):
                sl_hi = pl.ds(CHUNK + g * 16, 16)
                sl_lo = pl.ds(g * 16, 16)
                csrc[sl_lo] = csrc[sl_hi]
                cdoff[sl_lo] = cdoff[sl_hi]

        return jnp.where(off >= CHUNK, off - CHUNK, off)

    off = lax.fori_loop(0, CPT, _chunk, jnp.int32(0))

    @pl.when(off > 0)
    def _drain():
        _flush(True, off)

    plsc.subcore_barrier()

    def _copy_rows(soff, cnt):
        pltpu.sync_copy(num_sp.at[pl.ds(soff, cnt)], gbuf.at[pl.ds(0, cnt)])
        pltpu.sync_copy(gbuf.at[pl.ds(0, cnt)],
                        num_out.at[pl.ds(lo + soff, cnt)])

    tbase = sid * 1560
    _copy_rows(tbase, 400)
    _copy_rows(tbase + 400, 400)
    _copy_rows(tbase + 800, 400)
    _copy_rows(tbase + 1200, 360)

    @pl.when(sid == 0)
    def _tail():
        _copy_rows(16 * 1560, 40)


def _sc_edge_pass(src, dst, hx, adv, c16, width, acol):
    mesh = plsc.VectorSubcoreMesh(core_axis_name="c", subcore_axis_name="s")
    body = functools.partial(_sc_body, width, acol)

    @functools.partial(
        pl.kernel,
        mesh=mesh,
        compiler_params=_SC_PARAMS,
        out_type=jax.ShapeDtypeStruct((N, width), jnp.float32),
        scratch_types=[
            pltpu.VMEM((HALF,), jnp.float32),
            pltpu.VMEM((16,), jnp.float32),
            pltpu.VMEM((CHUNK,), jnp.int32),
            pltpu.VMEM((CHUNK,), jnp.int32),
            pltpu.VMEM((CHUNK,), jnp.float32),
            pltpu.VMEM((CBUF,), jnp.int32),
            pltpu.VMEM((CBUF,), jnp.int32),
            pltpu.VMEM((CHUNK, width), jnp.float32),
            pltpu.VMEM_SHARED((HALF_PAD, width), jnp.float32),
            pltpu.SemaphoreType.DMA,
            pltpu.SemaphoreType.DMA,
        ],
    )
    def k(src_hbm, dst_hbm, hx_hbm, ad_hbm, c_hbm, num_out,
          ad_t, c_v, srcb, dstb, exb, csrc, cdoff, gbuf, nsp, semg, sems):
        body(src_hbm, dst_hbm, hx_hbm, ad_hbm, c_hbm, num_out,
             ad_t, c_v, srcb, dstb, exb, csrc, cdoff, gbuf, nsp, semg, sems)

    return k(src, dst, hx, adv, c16)


def _layer(x, edge_src, edge_dst, W, att_s, att_d, b, relu):
    hxa, hxb, asv, adv, c = _dense(x, W, att_s, att_d)
    c16 = jnp.broadcast_to(c.reshape(()), (16,))
    adv1 = adv.reshape(N)
    numa = _sc_edge_pass(edge_src, edge_dst, hxa, adv1, c16, WA, DA + 1)
    numb = _sc_edge_pass(edge_src, edge_dst, hxb, adv1, c16, WB, DB + 1)
    return _finish(numa, numb, hxa, hxb, asv, adv, c, b, relu)


def kernel(t, x, edge_index, W1, as1, ad1, b1, W2, as2, ad2, b2,
           W3, as3, ad3, b3):
    src = edge_index[0]
    dst = edge_index[1]
    h = _layer(x, src, dst, W1, as1, ad1, b1, relu=True)
    h = _layer(h, src, dst, W2, as2, ad2, b2, relu=True)
    return _layer(h, src, dst, W3, as3, ad3, b3, relu=False)
